# baseline (speedup 1.0000x reference)
import functools

import jax
import jax.numpy as jnp
from jax import lax
from jax.experimental import pallas as pl
from jax.experimental.pallas import tpu as pltpu
from jax.experimental.pallas import tpu_sc as plsc

LAT = 64
XD = 16
UW = LAT + 2 * XD
IW = LAT + XD
B = 16384
NC, NS, L = 2, 16, 16
NW = NC * NS
RPW = B // NW
CHUNK = 128
NCH = RPW // CHUNK


def _sc_body(u_hbm, i_hbm, x_hbm, ucat_hbm, icat_hbm, out_hbm,
             uidx, iidx, xv, urows, irows, outv, sem):
  wid = lax.axis_index("s") * NC + lax.axis_index("c")
  base = wid * RPW

  pltpu.sync_copy(u_hbm.at[pl.ds(base, RPW)], uidx)
  pltpu.sync_copy(i_hbm.at[pl.ds(base, RPW)], iidx)
  pltpu.sync_copy(x_hbm.at[pl.ds(base, RPW)], xv)

  cps = []
  for j in range(NCH):
    rows = pl.ds(j * CHUNK, CHUNK)
    cps.append(pltpu.async_copy(ucat_hbm.at[uidx.at[rows]], urows.at[rows], sem))
    cps.append(pltpu.async_copy(icat_hbm.at[iidx.at[rows]], irows.at[rows], sem))
  for cp in cps:
    cp.wait()

  lane = lax.broadcasted_iota(jnp.int32, (L,), 0)

  @plsc.parallel_loop(0, RPW, 1, unroll=8)
  def row_body(r):
    rsplat = jnp.full((L,), r, jnp.int32)
    mfr = plsc.load_gather(xv, [rsplat]).astype(jnp.float32)
    u4 = urows[r, pl.ds(LAT, XD)] * mfr
    ux0 = urows[r, pl.ds(LAT + XD, XD)]
    i4 = ux0 + (irows[r, pl.ds(LAT, XD)] - ux0) * mfr
    ivec = [irows[r, pl.ds(j * L, L)] for j in range(4)] + [i4]
    uvec = [urows[r, pl.ds(j * L, L)] for j in range(4)] + [u4]
    mx = jnp.maximum(jnp.maximum(jnp.maximum(ivec[0], ivec[1]),
                                 jnp.maximum(ivec[2], ivec[3])), ivec[4])
    m_s = jnp.max(mx)
    e = [jnp.exp(v - m_s) for v in ivec]
    s = [1.0 / (1.0 + jnp.exp(-v)) for v in uvec]
    evec = (e[0] + e[1]) + (e[2] + e[3]) + e[4]
    pvec = (s[0] * e[0] + s[1] * e[1]) + (s[2] * e[2] + s[3] * e[3]) + s[4] * e[4]
    valvec = jnp.full((L,), jnp.sum(pvec), jnp.float32) / jnp.full(
        (L,), jnp.sum(evec), jnp.float32)
    plsc.store_scatter(outv, [rsplat], valvec, mask=lane == 0)

  del row_body
  pltpu.sync_copy(outv, out_hbm.at[pl.ds(base, RPW)])


RB = 512


def _tc_transpose_body(ut_ref, ux1_ref, ux0_ref, it_ref, ix1_ref,
                       ucat_ref, icat_ref):
  ucat_ref[:, pl.ds(0, LAT)] = jnp.transpose(ut_ref[...], (1, 0))
  ucat_ref[:, pl.ds(LAT, XD)] = jnp.transpose(ux1_ref[...], (1, 0))
  ucat_ref[:, pl.ds(LAT + XD, XD)] = jnp.transpose(ux0_ref[...], (1, 0))
  icat_ref[:, pl.ds(0, LAT)] = jnp.transpose(it_ref[...], (1, 0))
  icat_ref[:, pl.ds(LAT, XD)] = jnp.transpose(ix1_ref[...], (1, 0))


def _fuse_tables(ut_t, ux1_t, ux0_t, it_t, ix1_t):
  n = ut_t.shape[1]
  grid = (n + RB - 1) // RB
  wide = lambda w: pl.BlockSpec((w, RB), lambda i: (0, i))
  return pl.pallas_call(
      _tc_transpose_body,
      grid=(grid,),
      in_specs=[wide(LAT), wide(XD), wide(XD), wide(LAT), wide(XD)],
      out_specs=[pl.BlockSpec((RB, UW), lambda i: (i, 0)),
                 pl.BlockSpec((RB, IW), lambda i: (i, 0))],
      out_shape=[jax.ShapeDtypeStruct((n, UW), jnp.float32),
                 jax.ShapeDtypeStruct((n, IW), jnp.float32)],
  )(ut_t, ux1_t, ux0_t, it_t, ix1_t)


@jax.jit
def kernel(users, items, xij, user_table, item_table, item_xij1_table,
           item_xij0_table, user_xij1_table, user_xij0_table):
  del item_xij0_table
  ucat, icat = _fuse_tables(
      jnp.swapaxes(user_table, 0, 1), jnp.swapaxes(user_xij1_table, 0, 1),
      jnp.swapaxes(user_xij0_table, 0, 1), jnp.swapaxes(item_table, 0, 1),
      jnp.swapaxes(item_xij1_table, 0, 1))

  mesh = plsc.VectorSubcoreMesh(core_axis_name="c", subcore_axis_name="s")
  run = functools.partial(
      pl.kernel,
      out_type=jax.ShapeDtypeStruct((B,), jnp.float32),
      mesh=mesh,
      compiler_params=pltpu.CompilerParams(needs_layout_passes=False,
                                           use_tc_tiling_on_sc=False),
      scratch_types=[
          pltpu.VMEM((RPW,), jnp.int32),
          pltpu.VMEM((RPW,), jnp.int32),
          pltpu.VMEM((RPW,), jnp.int32),
          pltpu.VMEM((RPW, UW), jnp.float32),
          pltpu.VMEM((RPW, IW), jnp.float32),
          pltpu.VMEM((RPW,), jnp.float32),
          pltpu.SemaphoreType.DMA,
      ],
  )(_sc_body)
  return run(users.astype(jnp.int32), items.astype(jnp.int32),
             xij.astype(jnp.int32), ucat, icat)


# speedup vs baseline: 1.1637x; 1.1637x over previous
import functools

import jax
import jax.numpy as jnp
from jax import lax
from jax.experimental import pallas as pl
from jax.experimental.pallas import tpu as pltpu
from jax.experimental.pallas import tpu_sc as plsc

LAT = 64
XD = 16
UW = LAT + 2 * XD
IW = LAT + XD
B = 16384
NC, NS, L = 2, 16, 16
NW = NC * NS
RPW = B // NW
CHUNK = 128
NCH = RPW // CHUNK


def _sc_body(u_hbm, i_hbm, x_hbm, ucat_hbm, icat_hbm, out_hbm,
             uidx, iidx, xv, urows, irows, outv, sem):
  wid = lax.axis_index("s") * NC + lax.axis_index("c")
  base = wid * RPW

  pltpu.sync_copy(u_hbm.at[pl.ds(base, RPW)], uidx)
  pltpu.sync_copy(i_hbm.at[pl.ds(base, RPW)], iidx)
  pltpu.sync_copy(x_hbm.at[pl.ds(base, RPW)], xv)

  cps = []
  for j in range(NCH):
    rows = pl.ds(j * CHUNK, CHUNK)
    cps.append(pltpu.async_copy(ucat_hbm.at[uidx.at[rows]], urows.at[rows], sem))
    cps.append(pltpu.async_copy(icat_hbm.at[iidx.at[rows]], irows.at[rows], sem))
  for cp in cps:
    cp.wait()

  lane = lax.broadcasted_iota(jnp.int32, (L,), 0)

  @plsc.parallel_loop(0, RPW, 1, unroll=8)
  def row_body(r):
    rsplat = jnp.full((L,), r, jnp.int32)
    mfr = plsc.load_gather(xv, [rsplat]).astype(jnp.float32)
    u4 = urows[r, pl.ds(LAT, XD)] * mfr
    ux0 = urows[r, pl.ds(LAT + XD, XD)]
    i4 = ux0 + (irows[r, pl.ds(LAT, XD)] - ux0) * mfr
    ivec = [irows[r, pl.ds(j * L, L)] for j in range(4)] + [i4]
    uvec = [urows[r, pl.ds(j * L, L)] for j in range(4)] + [u4]
    mx = jnp.maximum(jnp.maximum(jnp.maximum(ivec[0], ivec[1]),
                                 jnp.maximum(ivec[2], ivec[3])), ivec[4])
    m_s = jnp.max(mx)
    e = [jnp.exp(v - m_s) for v in ivec]
    s = [1.0 / (1.0 + jnp.exp(-v)) for v in uvec]
    evec = (e[0] + e[1]) + (e[2] + e[3]) + e[4]
    pvec = (s[0] * e[0] + s[1] * e[1]) + (s[2] * e[2] + s[3] * e[3]) + s[4] * e[4]
    valvec = jnp.full((L,), jnp.sum(pvec), jnp.float32) / jnp.full(
        (L,), jnp.sum(evec), jnp.float32)
    plsc.store_scatter(outv, [rsplat], valvec, mask=lane == 0)

  del row_body
  pltpu.sync_copy(outv, out_hbm.at[pl.ds(base, RPW)])


RB = 1024


def _mxu_t(x):
  k = x.shape[0]
  i = lax.broadcasted_iota(jnp.int32, (k, k), 0)
  j = lax.broadcasted_iota(jnp.int32, (k, k), 1)
  eye = (i == j).astype(jnp.float32)
  return lax.dot_general(x, eye, (((0,), (0,)), ((), ())),
                         preferred_element_type=jnp.float32)


def _tc_transpose_body(ut_ref, ux1_ref, ux0_ref, it_ref, ix1_ref,
                       ucat_ref, icat_ref):
  ucat_ref[:, pl.ds(0, LAT)] = _mxu_t(ut_ref[...])
  ucat_ref[:, pl.ds(LAT, XD)] = _mxu_t(ux1_ref[...])
  ucat_ref[:, pl.ds(LAT + XD, XD)] = _mxu_t(ux0_ref[...])
  icat_ref[:, pl.ds(0, LAT)] = _mxu_t(it_ref[...])
  icat_ref[:, pl.ds(LAT, XD)] = _mxu_t(ix1_ref[...])


def _fuse_tables(ut_t, ux1_t, ux0_t, it_t, ix1_t):
  n = ut_t.shape[1]
  grid = (n + RB - 1) // RB
  wide = lambda w: pl.BlockSpec((w, RB), lambda i: (0, i))
  return pl.pallas_call(
      _tc_transpose_body,
      grid=(grid,),
      in_specs=[wide(LAT), wide(XD), wide(XD), wide(LAT), wide(XD)],
      out_specs=[pl.BlockSpec((RB, UW), lambda i: (i, 0)),
                 pl.BlockSpec((RB, IW), lambda i: (i, 0))],
      out_shape=[jax.ShapeDtypeStruct((n, UW), jnp.float32),
                 jax.ShapeDtypeStruct((n, IW), jnp.float32)],
  )(ut_t, ux1_t, ux0_t, it_t, ix1_t)


@jax.jit
def kernel(users, items, xij, user_table, item_table, item_xij1_table,
           item_xij0_table, user_xij1_table, user_xij0_table):
  del item_xij0_table
  ucat, icat = _fuse_tables(
      jnp.swapaxes(user_table, 0, 1), jnp.swapaxes(user_xij1_table, 0, 1),
      jnp.swapaxes(user_xij0_table, 0, 1), jnp.swapaxes(item_table, 0, 1),
      jnp.swapaxes(item_xij1_table, 0, 1))

  mesh = plsc.VectorSubcoreMesh(core_axis_name="c", subcore_axis_name="s")
  run = functools.partial(
      pl.kernel,
      out_type=jax.ShapeDtypeStruct((B,), jnp.float32),
      mesh=mesh,
      compiler_params=pltpu.CompilerParams(needs_layout_passes=False,
                                           use_tc_tiling_on_sc=False),
      scratch_types=[
          pltpu.VMEM((RPW,), jnp.int32),
          pltpu.VMEM((RPW,), jnp.int32),
          pltpu.VMEM((RPW,), jnp.int32),
          pltpu.VMEM((RPW, UW), jnp.float32),
          pltpu.VMEM((RPW, IW), jnp.float32),
          pltpu.VMEM((RPW,), jnp.float32),
          pltpu.SemaphoreType.DMA,
      ],
  )(_sc_body)
  return run(users.astype(jnp.int32), items.astype(jnp.int32),
             xij.astype(jnp.int32), ucat, icat)
